# baseline (speedup 1.0000x reference)
import functools

import jax
import jax.numpy as jnp
from jax import lax
from jax.experimental import pallas as pl
from jax.experimental.pallas import tpu as pltpu
from jax.experimental.pallas import tpu_sc as plsc

_MAX_ROWS = 8192
_OUT_ROWS = 4096
_D = 2048
_NC = 2
_NS = 16
_NW = _NC * _NS
_ROWS_PER_W = _OUT_ROWS // _NW

_mesh = plsc.VectorSubcoreMesh(
    core_axis_name="c", subcore_axis_name="s", num_cores=_NC, num_subcores=_NS
)


@functools.partial(
    pl.kernel,
    out_type=jax.ShapeDtypeStruct((_OUT_ROWS, _D), jnp.float32),
    mesh=_mesh,
    scratch_types=[pltpu.VMEM((16,), jnp.int32)],
)
def _sc_slice_copy(table_hbm, start_hbm, out_hbm, start_v):
    wid = lax.axis_index("s") * _NC + lax.axis_index("c")
    pltpu.sync_copy(start_hbm, start_v)
    s = lax.squeeze(lax.slice(start_v[...], (0,), (1,)), (0,))
    base = wid * _ROWS_PER_W
    src = pl.multiple_of((s + base) // 8 * 8, 8)
    pltpu.sync_copy(
        table_hbm.at[pl.ds(src, _ROWS_PER_W)],
        out_hbm.at[pl.ds(base, _ROWS_PER_W)],
    )


def kernel(position_codes, seq_length):
    start = jnp.clip(
        jnp.asarray(seq_length, jnp.int32) - _OUT_ROWS, 0, _MAX_ROWS - _OUT_ROWS
    )
    start_vec = jnp.full((16,), start, dtype=jnp.int32)
    return _sc_slice_copy(position_codes, start_vec)


# speedup vs baseline: 23.3055x; 23.3055x over previous
import functools

import jax
import jax.numpy as jnp
from jax import lax
from jax.experimental import pallas as pl
from jax.experimental.pallas import tpu as pltpu
from jax.experimental.pallas import tpu_sc as plsc

_MAX_ROWS = 8192
_OUT_ROWS = 4096
_D = 2048
_NC = 2
_NS = 16
_NW = _NC * _NS
_ROWS_PER_W = _OUT_ROWS // _NW
_CHUNK = 16
_NCHUNK = _ROWS_PER_W // _CHUNK

_mesh = plsc.VectorSubcoreMesh(
    core_axis_name="c", subcore_axis_name="s", num_cores=_NC, num_subcores=_NS
)


@functools.partial(
    pl.kernel,
    out_type=jax.ShapeDtypeStruct((_OUT_ROWS, _D), jnp.float32),
    mesh=_mesh,
    scratch_types=[
        pltpu.VMEM((16,), jnp.int32),
        pltpu.VMEM((_CHUNK, _D), jnp.float32),
        pltpu.VMEM((_CHUNK, _D), jnp.float32),
        pltpu.SemaphoreType.DMA,
        pltpu.SemaphoreType.DMA,
        pltpu.SemaphoreType.DMA,
        pltpu.SemaphoreType.DMA,
    ],
)
def _sc_slice_copy(table_hbm, start_hbm, out_hbm, start_v, buf0, buf1,
                   gsem0, gsem1, ssem0, ssem1):
    wid = lax.axis_index("s") * _NC + lax.axis_index("c")
    pltpu.sync_copy(start_hbm, start_v)
    s = lax.squeeze(lax.slice(start_v[...], (0,), (1,)), (0,))
    base = wid * _ROWS_PER_W
    bufs = (buf0, buf1)
    gsems = (gsem0, gsem1)
    ssems = (ssem0, ssem1)
    scat = [None, None]
    for g in range(_NCHUNK):
        b = g & 1
        src = pl.multiple_of((s + base + g * _CHUNK) // 8 * 8, 8)
        dst = pl.multiple_of(base + g * _CHUNK, 8)
        if scat[b] is not None:
            scat[b].wait()
        gath = pltpu.make_async_copy(
            table_hbm.at[pl.ds(src, _CHUNK)], bufs[b], gsems[b]
        )
        gath.start()
        gath.wait()
        scat[b] = pltpu.make_async_copy(
            bufs[b], out_hbm.at[pl.ds(dst, _CHUNK)], ssems[b]
        )
        scat[b].start()
    for h in scat:
        if h is not None:
            h.wait()


def kernel(position_codes, seq_length):
    start = jnp.clip(
        jnp.asarray(seq_length, jnp.int32) - _OUT_ROWS, 0, _MAX_ROWS - _OUT_ROWS
    )
    start_vec = jnp.full((16,), start, dtype=jnp.int32)
    return _sc_slice_copy(position_codes, start_vec)


# speedup vs baseline: 23.4867x; 1.0078x over previous
import functools

import jax
import jax.numpy as jnp
from jax import lax
from jax.experimental import pallas as pl
from jax.experimental.pallas import tpu as pltpu
from jax.experimental.pallas import tpu_sc as plsc

_MAX_ROWS = 8192
_OUT_ROWS = 4096
_D = 2048
_NC = 2
_NS = 16
_NW = _NC * _NS
_ROWS_PER_W = _OUT_ROWS // _NW
_CHUNK = 16
_NCHUNK = _ROWS_PER_W // _CHUNK
_NBUF = 3

_mesh = plsc.VectorSubcoreMesh(
    core_axis_name="c", subcore_axis_name="s", num_cores=_NC, num_subcores=_NS
)


@functools.partial(
    pl.kernel,
    out_type=jax.ShapeDtypeStruct((_OUT_ROWS, _D), jnp.float32),
    mesh=_mesh,
    scratch_types=[
        pltpu.VMEM((16,), jnp.int32),
        [pltpu.VMEM((_CHUNK, _D), jnp.float32)] * _NBUF,
        [pltpu.SemaphoreType.DMA] * _NBUF,
        [pltpu.SemaphoreType.DMA] * _NBUF,
    ],
)
def _sc_slice_copy(table_hbm, seq_hbm, out_hbm, seq_v, bufs, gsems, ssems):
    wid = lax.axis_index("s") * _NC + lax.axis_index("c")
    pltpu.sync_copy(seq_hbm, seq_v)
    seq = seq_v[...]
    start = jnp.minimum(
        jnp.maximum(seq - _OUT_ROWS, 0), _MAX_ROWS - _OUT_ROWS
    )
    s = lax.squeeze(lax.slice(start, (0,), (1,)), (0,))
    base = wid * _ROWS_PER_W
    scat = [None] * _NBUF
    for g in range(_NCHUNK):
        b = g % _NBUF
        src = pl.multiple_of((s + base + g * _CHUNK) // 8 * 8, 8)
        dst = pl.multiple_of(base + g * _CHUNK, 8)
        if scat[b] is not None:
            scat[b].wait()
        gath = pltpu.make_async_copy(
            table_hbm.at[pl.ds(src, _CHUNK)], bufs[b], gsems[b]
        )
        gath.start()
        gath.wait()
        scat[b] = pltpu.make_async_copy(
            bufs[b], out_hbm.at[pl.ds(dst, _CHUNK)], ssems[b]
        )
        scat[b].start()
    for h in scat:
        if h is not None:
            h.wait()


def kernel(position_codes, seq_length):
    seq_vec = jnp.full((16,), seq_length, dtype=jnp.int32)
    return _sc_slice_copy(position_codes, seq_vec)
